# baseline (speedup 1.0000x reference)
import functools

import jax
import jax.numpy as jnp
from jax import lax
from jax.experimental import pallas as pl
from jax.experimental.pallas import tpu as pltpu
from jax.experimental.pallas import tpu_sc as plsc

_RATE = 0.75
_MIN32 = -(2**31)


def _sc_body(k, n, chunk, imp_hbm, mask_hbm, cnt_hbm, buf, obuf, hist, cbuf):
    b = lax.axis_index("s") * 2 + lax.axis_index("c")
    lane = lax.iota(jnp.int32, 16)
    ones = jnp.ones((16,), jnp.int32)
    nch = n // chunk
    nvec = chunk // 16

    def keys_at(j):
        x = buf[pl.ds(j * 16, 16)]
        i32 = lax.bitcast_convert_type(x, jnp.int32)
        return i32 ^ ((i32 >> 31) & jnp.int32(0x7FFFFFFF))

    prefix = jnp.int32(0)
    k_rem = jnp.int32(k)
    for p in range(4):
        shift_b = 24 - 8 * p

        def zero_body(i, _):
            hist[i] = jnp.zeros((16,), jnp.int32)
            return 0

        lax.fori_loop(0, 256, zero_body, 0)

        for c in range(nch):
            pltpu.sync_copy(imp_hbm.at[b, pl.ds(c * chunk, chunk)], buf)

            def vec_body(j, _, p=p, shift_b=shift_b, prefix=prefix):
                ukey = keys_at(j) ^ jnp.int32(_MIN32)
                if shift_b:
                    bucket = lax.shift_right_logical(ukey, shift_b) & 0xFF
                else:
                    bucket = ukey & 0xFF
                if p == 0:
                    plsc.addupdate_scatter(hist, [bucket, lane], ones)
                else:
                    hi = lax.shift_right_logical(ukey, shift_b + 8)
                    plsc.addupdate_scatter(
                        hist, [bucket, lane], ones, mask=hi == prefix
                    )
                return 0

            lax.fori_loop(0, nvec, vec_body, 0)

        def scan_body(i, carry):
            cum, bstar, cabove = carry
            bi = 255 - i
            s = jnp.sum(hist[bi])
            newcum = cum + s
            hit = (cum < k_rem) & (newcum >= k_rem)
            return (
                newcum,
                jnp.where(hit, bi, bstar),
                jnp.where(hit, cum, cabove),
            )

        _, bstar, cabove = lax.fori_loop(
            0, 256, scan_body, (jnp.int32(0), jnp.int32(0), jnp.int32(0))
        )
        prefix = (prefix << 8) | bstar
        k_rem = k_rem - cabove

    t_key = prefix ^ jnp.int32(_MIN32)
    need = k_rem

    rank_c = jnp.zeros((16,), jnp.int32)
    cnt_c = jnp.zeros((16,), jnp.int32)
    for c in range(nch):
        pltpu.sync_copy(imp_hbm.at[b, pl.ds(c * chunk, chunk)], buf)

        def mask_body(j, carry):
            rank_c, cnt_c = carry
            key = keys_at(j)
            gt = key > t_key
            tie = key == t_key
            cs = plsc.cumsum(jnp.where(tie, jnp.int32(1), jnp.int32(0)))
            keep = gt | (tie & ((cs + rank_c) <= need))
            obuf[pl.ds(j * 16, 16)] = jnp.where(
                keep, jnp.float32(1.0), jnp.float32(0.0)
            )
            return (
                rank_c + plsc.all_reduce_population_count(tie),
                cnt_c + plsc.all_reduce_population_count(keep),
            )

        rank_c, cnt_c = lax.fori_loop(0, nvec, mask_body, (rank_c, cnt_c))
        pltpu.sync_copy(obuf, mask_hbm.at[b, pl.ds(c * chunk, chunk)])

    cbuf[pl.ds(0, 16)] = cnt_c.astype(jnp.float32)
    pltpu.sync_copy(cbuf, cnt_hbm.at[b])


@functools.partial(jax.jit, static_argnums=())
def kernel(imp):
    B, H, W = imp.shape
    n = H * W
    k = max(1, int(_RATE * n))
    chunk = 16384
    mesh = plsc.VectorSubcoreMesh(core_axis_name="c", subcore_axis_name="s")
    sc_call = pl.kernel(
        functools.partial(_sc_body, k, n, chunk),
        out_type=[
            jax.ShapeDtypeStruct((B, n), jnp.float32),
            jax.ShapeDtypeStruct((B, 16), jnp.float32),
        ],
        mesh=mesh,
        compiler_params=pltpu.CompilerParams(needs_layout_passes=False),
        scratch_types=[
            pltpu.VMEM((chunk,), jnp.float32),
            pltpu.VMEM((chunk,), jnp.float32),
            pltpu.VMEM((256, 16), jnp.int32),
            pltpu.VMEM((16,), jnp.float32),
        ],
    )
    mask2d, cnt = sc_call(imp.reshape(B, n))
    mean = jnp.sum(cnt[:, 0]) / jnp.float32(B * n)
    return mask2d.reshape(B, 1, H, W), mean
